# Initial kernel scaffold; baseline (speedup 1.0000x reference)
import jax
import jax.numpy as jnp
from jax.experimental import pallas as pl


def kernel(heatmap, vmin_s1, vmax, vmin):
    raise NotImplementedError("write your pallas kernel here")


# speedup vs baseline: 3739.3474x; 3739.3474x over previous
import functools

import jax
import jax.numpy as jnp
from jax import lax
from jax.experimental import pallas as pl
from jax.experimental.pallas import tpu as pltpu

BOUND_OFF = 3.0
_PREC = jax.lax.Precision.HIGHEST


def _stats_kernel(h_ref, t_ref, vb_ref, oh_ref, idxr_ref, bd_ref, *, J, V, G):
    b = pl.program_id(0)
    fmax = float(V - 1)
    pxs, pys, pzs = [], [], []
    ii = lax.broadcasted_iota(jnp.int32, (V, V, V), 0).astype(jnp.float32)
    jj = lax.broadcasted_iota(jnp.int32, (V, V, V), 1).astype(jnp.float32)
    kk = lax.broadcasted_iota(jnp.int32, (V, V, V), 2).astype(jnp.float32)
    for j in range(J):
        hj = h_ref[0, j]
        mx = jnp.max(hj)
        mask = (hj == mx).astype(jnp.float32)
        cnt = jnp.sum(mask)
        pxs.append(jnp.sum(mask * ii) / cnt)
        pys.append(jnp.sum(mask * jj) / cnt)
        pzs.append(jnp.sum(mask * kk) / cnt)

    def _build(axis, plist):
        mn = functools.reduce(jnp.minimum, plist)
        mxp = functools.reduce(jnp.maximum, plist)
        max_bd = jnp.clip(mxp + BOUND_OFF, 0.0, fmax)
        min_bd = jnp.clip(mn - BOUND_OFF, 0.0, fmax)
        vmin_a = vb_ref[b, axis]
        vmax_a = vb_ref[b, 3 + axis]
        interval = (vmax_a - vmin_a) / fmax
        max_b = vmin_a + max_bd / fmax * (vmax_a - vmin_a)
        min_b = vmin_a + min_bd / fmax * (vmax_a - vmin_a)
        return min_b, max_b, vmin_a, interval

    mnx, mxx, vminx, intx = _build(0, pxs)
    mny, mxy, vminy, inty = _build(1, pys)
    mnz, mxz, vminz, intz = _build(2, pzs)

    t32 = jnp.broadcast_to(t_ref[0:1, :], (V, G))
    vlane = lax.broadcasted_iota(jnp.int32, (V, G), 0).astype(jnp.float32)
    for slot, (mn, mxw, vmin_a, inter) in enumerate(
        [(mny, mxy, vminy, inty), (mnz, mxz, vminz, intz)]
    ):
        g = mn + t32 * (mxw - mn)
        vox = (g - vmin_a) / inter
        idxf = jnp.clip(jnp.round(vox), 0.0, fmax)
        oh_ref[0, slot] = (idxf == vlane).astype(jnp.float32)

    gx = mnx + t_ref[...] * (mxx - mnx)
    voxx = (gx - vminx) / intx
    idxr_ref[0] = jnp.clip(jnp.round(voxx), 0.0, fmax)

    lane = lax.broadcasted_iota(jnp.int32, (8, 128), 1)
    bd = jnp.zeros((8, 128), jnp.float32)
    for p, val in enumerate([mnx, mny, mnz, mxx, mxy, mxz]):
        bd = jnp.where(lane == p, val, bd)
    bd_ref[0] = bd


def _expand_kernel(h_ref, oh_ref, idx_ref, out_ref, sc_ref, *, V, G):
    b = pl.program_id(0)
    hv = h_ref[0, 0]
    ht = jnp.transpose(hv, (0, 2, 1))
    a = ht.reshape(V * V, V)
    sy = oh_ref[0, 0]
    sz = oh_ref[0, 1]
    b1 = jax.lax.dot(a, sy, precision=_PREC,
                     preferred_element_type=jnp.float32)
    b3 = jnp.transpose(b1.reshape(V, V, G), (0, 2, 1))
    a2 = b3.reshape(V * G, V)
    c = jax.lax.dot(a2, sz, precision=_PREC,
                    preferred_element_type=jnp.float32)
    sc_ref[...] = c.reshape(V, G, G)

    def body(x, carry):
        ix = idx_ref[b, x]
        out_ref[0, 0, pl.ds(x, 1)] = sc_ref[pl.ds(ix, 1)]
        return carry

    lax.fori_loop(0, G, body, 0)


def kernel(heatmap, vmin_s1, vmax, vmin):
    B, J, V = heatmap.shape[0], heatmap.shape[1], heatmap.shape[2]
    G = 2 * V
    t = jnp.linspace(0.0, 1.0, G).astype(jnp.float32)
    t_row = jnp.broadcast_to(t[None, :], (8, G))
    vb = jnp.concatenate(
        [vmin[:, 0, :], vmax[:, 0, :], jnp.zeros((B, 2), jnp.float32)], axis=1)

    oh, idxr, bd = pl.pallas_call(
        functools.partial(_stats_kernel, J=J, V=V, G=G),
        grid=(B,),
        in_specs=[
            pl.BlockSpec((1, J, V, V, V), lambda b: (b, 0, 0, 0, 0)),
            pl.BlockSpec((8, G), lambda b: (0, 0)),
            pl.BlockSpec(memory_space=pltpu.SMEM),
        ],
        out_specs=[
            pl.BlockSpec((1, 2, V, G), lambda b: (b, 0, 0, 0)),
            pl.BlockSpec((1, 8, G), lambda b: (b, 0, 0)),
            pl.BlockSpec((1, 8, 128), lambda b: (b, 0, 0)),
        ],
        out_shape=[
            jax.ShapeDtypeStruct((B, 2, V, G), jnp.float32),
            jax.ShapeDtypeStruct((B, 8, G), jnp.float32),
            jax.ShapeDtypeStruct((B, 8, 128), jnp.float32),
        ],
    )(heatmap, t_row, vb)

    idx_x = idxr[:, 0, :].astype(jnp.int32)

    interp = pl.pallas_call(
        functools.partial(_expand_kernel, V=V, G=G),
        grid=(B, J),
        in_specs=[
            pl.BlockSpec((1, 1, V, V, V), lambda b, j: (b, j, 0, 0, 0)),
            pl.BlockSpec((1, 2, V, G), lambda b, j: (b, 0, 0, 0)),
            pl.BlockSpec(memory_space=pltpu.SMEM),
        ],
        out_specs=pl.BlockSpec((1, 1, G, G, G), lambda b, j: (b, j, 0, 0, 0)),
        out_shape=jax.ShapeDtypeStruct((B, J, G, G, G), jnp.float32),
        scratch_shapes=[pltpu.VMEM((V, G, G), jnp.float32)],
    )(heatmap, oh, idx_x)

    min_b = bd[:, 0, 0:3].reshape(B, 1, 3)
    max_b = bd[:, 0, 3:6].reshape(B, 1, 3)
    return interp, max_b, min_b
